# Initial kernel scaffold; baseline (speedup 1.0000x reference)
import jax
import jax.numpy as jnp
from jax.experimental import pallas as pl


def kernel(x, weight):
    raise NotImplementedError("write your pallas kernel here")


# speedup vs baseline: 2.4487x; 2.4487x over previous
import functools

import jax
import jax.numpy as jnp
from jax import lax
from jax.experimental import pallas as pl
from jax.experimental.pallas import tpu as pltpu
from jax.experimental.pallas import tpu_sc as plsc

NUM_ROWS = 1_000_000
DIM = 64
BATCH = 16384
QMAX = 127.0
RNE_MAGIC = 1.5 * 2.0**23

NC = 2
NS = 16
NW = NC * NS
B_PER_W = BATCH // NW
IDX_CHUNK = 128
N_CHUNKS = B_PER_W // IDX_CHUNK

C = 32_768
H = C // 2
LOG2H = 14
ABS_GRID = -(-NUM_ROWS // C)
PACKED_ROWS = ABS_GRID * H


def _absmax_pack_body(x_ref, w_ref, scale_ref, si_ref, p_ref, pack_ref,
                      acc_ref):
    i = pl.program_id(0)

    @pl.when(i == 0)
    def _init():
        acc_ref[0] = 0.0
        v = x_ref[...]
        p_ref[...] = (v & ~(C - 1)) | ((v & (H - 1)) << 1) | ((v >> LOG2H) & 1)

    w = w_ref[...]
    u = jnp.concatenate([w[:, :H], w[:, H:]], axis=0)
    pack_ref[...] = u.T

    @pl.when(i < ABS_GRID - 1)
    def _full():
        acc_ref[0] = jnp.maximum(acc_ref[0], jnp.max(jnp.abs(w)))

    @pl.when(i == ABS_GRID - 1)
    def _masked_tail():
        col = i * C + jax.lax.broadcasted_iota(jnp.int32, (DIM, C), 1)
        a = jnp.where(col < NUM_ROWS, jnp.abs(w), 0.0)
        m = jnp.maximum(acc_ref[0], jnp.max(a))
        s = jnp.maximum(m, 1e-8) / QMAX
        scale_ref[0] = s
        inv = 1.0 / s
        for l in range(16):
            si_ref[l] = s
            si_ref[16 + l] = inv


_absmax_pack = pl.pallas_call(
    _absmax_pack_body,
    grid=(ABS_GRID,),
    in_specs=[
        pl.BlockSpec((NW, N_CHUNKS, IDX_CHUNK), lambda i: (0, 0, 0)),
        pl.BlockSpec((DIM, C), lambda i: (0, i)),
    ],
    out_specs=[
        pl.BlockSpec(memory_space=pltpu.SMEM),
        pl.BlockSpec(memory_space=pltpu.SMEM),
        pl.BlockSpec((NW, N_CHUNKS, IDX_CHUNK), lambda i: (0, 0, 0)),
        pl.BlockSpec((H, 128), lambda i: (i, 0)),
    ],
    out_shape=[
        jax.ShapeDtypeStruct((1,), jnp.float32),
        jax.ShapeDtypeStruct((32,), jnp.float32),
        jax.ShapeDtypeStruct((NW, N_CHUNKS, IDX_CHUNK), jnp.int32),
        jax.ShapeDtypeStruct((PACKED_ROWS, 128), jnp.float32),
    ],
    scratch_shapes=[pltpu.SMEM((1,), jnp.float32)],
)


@functools.cache
def _make_sc_gather():
    @functools.partial(
        pl.kernel,
        mesh=plsc.VectorSubcoreMesh(core_axis_name="c", subcore_axis_name="s"),
        compiler_params=pltpu.CompilerParams(use_tc_tiling_on_sc=False),
        out_type=jax.ShapeDtypeStruct((BATCH, DIM), jnp.float32),
        scratch_types=[
            pltpu.VMEM((N_CHUNKS, IDX_CHUNK), jnp.int32),
            pltpu.VMEM((B_PER_W, DIM), jnp.float32),
            pltpu.VMEM((32,), jnp.float32),
            pltpu.SemaphoreType.DMA,
        ],
    )
    def _sc_gather(p_hbm, table_hbm, si_hbm, out_hbm, p_v, rows_v, si_v, sem):
        wid = lax.axis_index("s") * NC + lax.axis_index("c")
        pltpu.sync_copy(si_hbm, si_v)
        pltpu.sync_copy(p_hbm.at[wid], p_v)
        s = si_v[pl.ds(0, 16)]
        inv = si_v[pl.ds(16, 16)]
        copies = [
            pltpu.async_copy(
                table_hbm.at[p_v.at[j]],
                rows_v.at[pl.ds(j * IDX_CHUNK, IDX_CHUNK)],
                sem,
            )
            for j in range(N_CHUNKS)
        ]

        def _dequant_row(r, carry):
            for k in range(DIM // 16):
                a = rows_v[r, pl.ds(k * 16, 16)] * inv
                q = (a + RNE_MAGIC) - RNE_MAGIC
                q = jnp.minimum(jnp.maximum(q, -QMAX), QMAX - 1.0)
                rows_v[r, pl.ds(k * 16, 16)] = q * s
            return carry

        for j in range(N_CHUNKS):
            copies[j].wait()
            lax.fori_loop(j * IDX_CHUNK, (j + 1) * IDX_CHUNK, _dequant_row, 0)
        pltpu.sync_copy(rows_v, out_hbm.at[pl.ds(wid * B_PER_W, B_PER_W)])

    return _sc_gather


def kernel(x, weight):
    x3 = x.reshape(NW, N_CHUNKS, IDX_CHUNK)
    scale, si, p3, packed = _absmax_pack(x3, weight.T)
    table = packed.reshape(2 * PACKED_ROWS, DIM)
    emb = _make_sc_gather()(p3, table, si)
    return emb, scale


# speedup vs baseline: 2.4612x; 1.0051x over previous
import functools

import jax
import jax.numpy as jnp
from jax import lax
from jax.experimental import pallas as pl
from jax.experimental.pallas import tpu as pltpu
from jax.experimental.pallas import tpu_sc as plsc

NUM_ROWS = 1_000_000
DIM = 64
BATCH = 16384
QMAX = 127.0
RNE_MAGIC = 1.5 * 2.0**23

NC = 2
NS = 16
NW = NC * NS
B_PER_W = BATCH // NW
IDX_CHUNK = 128
N_CHUNKS = B_PER_W // IDX_CHUNK

C = 32_768
H = C // 2
LOG2H = 14
ABS_GRID = -(-NUM_ROWS // C)
PACKED_ROWS = ABS_GRID * H


def _absmax_pack_body(x_ref, w_ref, scale_ref, si_ref, p_ref, pack_ref,
                      acc_ref):
    i = pl.program_id(0)

    @pl.when(i == 0)
    def _init():
        acc_ref[0] = 0.0
        v = x_ref[...]
        p_ref[...] = (v & ~(C - 1)) | ((v & (H - 1)) << 1) | ((v >> LOG2H) & 1)

    w = w_ref[...]
    u = jnp.concatenate([w[:, :H], w[:, H:]], axis=0)
    pack_ref[...] = u.T

    @pl.when(i < ABS_GRID - 1)
    def _full():
        acc_ref[0] = jnp.maximum(acc_ref[0], jnp.max(jnp.abs(w)))

    @pl.when(i == ABS_GRID - 1)
    def _masked_tail():
        col = i * C + jax.lax.broadcasted_iota(jnp.int32, (DIM, C), 1)
        a = jnp.where(col < NUM_ROWS, jnp.abs(w), 0.0)
        m = jnp.maximum(acc_ref[0], jnp.max(a))
        s = jnp.maximum(m, 1e-8) / QMAX
        scale_ref[0] = s
        inv = 1.0 / s
        for l in range(16):
            si_ref[l] = s
            si_ref[16 + l] = inv


_absmax_pack = pl.pallas_call(
    _absmax_pack_body,
    grid=(ABS_GRID,),
    in_specs=[
        pl.BlockSpec((NW, N_CHUNKS, IDX_CHUNK), lambda i: (0, 0, 0)),
        pl.BlockSpec((DIM, C), lambda i: (0, i)),
    ],
    out_specs=[
        pl.BlockSpec(memory_space=pltpu.SMEM),
        pl.BlockSpec(memory_space=pltpu.SMEM),
        pl.BlockSpec((NW, N_CHUNKS, IDX_CHUNK), lambda i: (0, 0, 0)),
        pl.BlockSpec((H, 128), lambda i: (i, 0)),
    ],
    out_shape=[
        jax.ShapeDtypeStruct((1,), jnp.float32),
        jax.ShapeDtypeStruct((32,), jnp.float32),
        jax.ShapeDtypeStruct((NW, N_CHUNKS, IDX_CHUNK), jnp.int32),
        jax.ShapeDtypeStruct((PACKED_ROWS, 128), jnp.float32),
    ],
    scratch_shapes=[pltpu.SMEM((1,), jnp.float32)],
)


@functools.cache
def _make_sc_gather():
    @functools.partial(
        pl.kernel,
        mesh=plsc.VectorSubcoreMesh(core_axis_name="c", subcore_axis_name="s"),
        compiler_params=pltpu.CompilerParams(use_tc_tiling_on_sc=False),
        out_type=jax.ShapeDtypeStruct((BATCH, DIM), jnp.float32),
        scratch_types=[
            pltpu.VMEM((N_CHUNKS, IDX_CHUNK), jnp.int32),
            pltpu.VMEM((B_PER_W, DIM), jnp.float32),
            pltpu.VMEM((32,), jnp.float32),
            pltpu.SemaphoreType.DMA,
        ],
    )
    def _sc_gather(p_hbm, table_hbm, si_hbm, out_hbm, p_v, rows_v, si_v, sem):
        wid = lax.axis_index("s") * NC + lax.axis_index("c")
        pltpu.sync_copy(si_hbm, si_v)
        pltpu.sync_copy(p_hbm.at[wid], p_v)
        s = si_v[pl.ds(0, 16)]
        inv = si_v[pl.ds(16, 16)]
        copies = [
            pltpu.async_copy(
                table_hbm.at[p_v.at[j]],
                rows_v.at[pl.ds(j * IDX_CHUNK, IDX_CHUNK)],
                sem,
            )
            for j in range(N_CHUNKS)
        ]

        def _dequant_row(r, carry):
            for k in range(DIM // 16):
                a = rows_v[r, pl.ds(k * 16, 16)] * inv
                q = (a + RNE_MAGIC) - RNE_MAGIC
                q = jnp.minimum(jnp.maximum(q, -QMAX), QMAX - 1.0)
                rows_v[r, pl.ds(k * 16, 16)] = q * s
            return carry

        stores = []
        for j in range(N_CHUNKS):
            copies[j].wait()
            lax.fori_loop(j * IDX_CHUNK, (j + 1) * IDX_CHUNK, _dequant_row, 0)
            stores.append(pltpu.async_copy(
                rows_v.at[pl.ds(j * IDX_CHUNK, IDX_CHUNK)],
                out_hbm.at[pl.ds(wid * B_PER_W + j * IDX_CHUNK, IDX_CHUNK)],
                sem,
            ))
        for st in stores:
            st.wait()

    return _sc_gather


def kernel(x, weight):
    x3 = x.reshape(NW, N_CHUNKS, IDX_CHUNK)
    scale, si, p3, packed = _absmax_pack(x3, weight.T)
    table = packed.reshape(2 * PACKED_ROWS, DIM)
    emb = _make_sc_gather()(p3, table, si)
    return emb, scale
